# baseline (speedup 1.0000x reference)
import functools

import jax
import jax.numpy as jnp
from jax import lax
from jax.experimental import pallas as pl
from jax.experimental.pallas import tpu as pltpu
from jax.experimental.pallas import tpu_sc as plsc

_T = 2048
_B = 64
_N = 256
_K = 16
_DT = 0.001

_TBLK = 32
_NBLK = 3
_SB = 16
_TP = _NBLK * _TBLK
_TB = 32
_NG = 32
_KP = 32


def _make_decoder():
    mesh = plsc.VectorSubcoreMesh(core_axis_name="c", subcore_axis_name="s")

    @functools.partial(
        pl.kernel,
        out_type=jax.ShapeDtypeStruct((_B, _KP, _N), jnp.float32),
        mesh=mesh,
        scratch_types=[
            pltpu.VMEM((_TP, 8, 128), jnp.float32),
            pltpu.VMEM((4, _KP, 128), jnp.float32),
            pltpu.VMEM((_NG, 16), jnp.int32),
            pltpu.SemaphoreType.DMA,
            pltpu.SemaphoreType.DMA,
            pltpu.SemaphoreType.DMA,
        ],
        compiler_params=pltpu.CompilerParams(needs_layout_passes=False),
    )
    def decode(x_hbm, out_hbm, chunk_v, out_v, cnt_v, sem0, sem1, sem2):
        core = lax.axis_index("c")
        sub = lax.axis_index("s")
        b0 = pl.multiple_of((sub % 8) * 8, 8)
        n0 = pl.multiple_of((sub // 8) * 128, 128)
        bh = core * 4

        sems = [sem0, sem1, sem2]
        copies = [
            pltpu.async_copy(
                x_hbm.at[pl.ds(blk * _TBLK, _TBLK),
                         pl.ds(b0, 8), pl.ds(n0, 128)],
                chunk_v.at[pl.ds(blk * _TBLK, _TBLK)],
                sems[blk])
            for blk in range(_NBLK)
        ]

        lane = jnp.arange(16, dtype=jnp.int32)
        inf_v = jnp.full((16,), jnp.inf, dtype=jnp.float32)
        one_v = jnp.ones((16,), dtype=jnp.int32)
        zero_v = jnp.zeros((16,), dtype=jnp.int32)
        k_v = jnp.full((16,), _K, dtype=jnp.int32)
        onef_v = jnp.ones((16,), dtype=jnp.float32)

        def raw_step(row, geom, cnt, tv):
            b_loc, n_off, b_rel_v, n_idx = geom
            v = chunk_v[row, b_loc, pl.ds(n_off, 16)]
            spike = v > 0.0
            plsc.store_scatter(out_v, [b_rel_v, cnt, n_idx], tv, mask=spike)
            return cnt + jnp.where(spike, one_v, zero_v)

        def num_live(cnt):
            return plsc.all_reduce_population_count(cnt < _K)[0]

        def group_geom(g):
            b_rel = g // 8
            n_off = (g % 8) * 16
            b_loc = bh + b_rel
            b_rel_v = jnp.broadcast_to(b_rel, (16,)).astype(jnp.int32)
            n_idx = n_off + lane
            return b_loc, n_off, b_rel_v, n_idx

        def scan_sub(t0, geom0, geom1, c0, c1, tv):
            @plsc.parallel_loop(t0, t0 + _SB, unroll=_SB, carry=(c0, c1, tv))
            def scan(row, state):
                c0, c1, tv = state
                c0 = raw_step(row, geom0, c0, tv)
                c1 = raw_step(row, geom1, c1, tv)
                return c0, c1, tv + onef_v

            c0, c1, tv = scan
            return jnp.minimum(c0, k_v), jnp.minimum(c1, k_v), tv

        mask = jnp.int32(-1)
        for blk in range(_NBLK):
            copies[blk].wait()

            def pair_body(p, mk, _blk=blk):
                g0 = p * 2
                g1 = g0 + 1

                def live(mk):
                    geom0 = group_geom(g0)
                    geom1 = group_geom(g1)
                    if _blk == 0:
                        c0 = zero_v
                        c1 = zero_v
                    else:
                        c0 = cnt_v[g0, :]
                        c1 = cnt_v[g1, :]
                    t0 = jnp.int32(_blk * _TBLK)
                    tv = jnp.broadcast_to(t0.astype(jnp.float32), (16,))

                    def sub_cond(state):
                        t, c0, c1, tv = state
                        return ((t < (_blk + 1) * _TBLK)
                                & (num_live(c0) + num_live(c1) > 0))

                    def sub_body(state):
                        t, c0, c1, tv = state
                        c0, c1, tv = scan_sub(t, geom0, geom1, c0, c1, tv)
                        return t + _SB, c0, c1, tv

                    _, c0, c1, _ = lax.while_loop(
                        sub_cond, sub_body, (t0, c0, c1, tv))
                    cnt_v[g0, :] = c0
                    cnt_v[g1, :] = c1
                    s0 = (num_live(c0) > 0).astype(jnp.int32)
                    s1 = (num_live(c1) > 0).astype(jnp.int32)
                    keep = ~((jnp.int32(1) << g0) | (jnp.int32(1) << g1))
                    return (mk & keep) | (s0 << g0) | (s1 << g1)

                return lax.cond((mk >> g0) & 3 != 0, live, lambda m: m, mk)

            mask = lax.fori_loop(0, _NG // 2, pair_body, mask)

        def rare_cond(state):
            t, mask = state
            return (t < _T) & (mask != 0)

        def rare_body(state):
            t, mask = state
            pltpu.sync_copy(
                x_hbm.at[pl.ds(t, _TB), pl.ds(b0, 8), pl.ds(n0, 128)],
                chunk_v.at[pl.ds(0, _TB)])

            def gb(g, mk):
                def live(mk):
                    geom = group_geom(g)
                    tv0 = jnp.broadcast_to(t.astype(jnp.float32), (16,))

                    def inner(i, state):
                        cnt, tv = state
                        v = chunk_v[i, geom[0], pl.ds(geom[1], 16)]
                        spike = v > 0.0
                        plsc.store_scatter(out_v, [geom[2], cnt, geom[3]],
                                           tv, mask=spike)
                        cnt = jnp.minimum(
                            cnt + jnp.where(spike, one_v, zero_v), k_v)
                        return cnt, tv + onef_v

                    cnt, _ = lax.fori_loop(0, _TB, inner, (cnt_v[g, :], tv0))
                    cnt_v[g, :] = cnt
                    done = num_live(cnt) == 0
                    return mk & ~jnp.where(done, jnp.int32(1) << g,
                                           jnp.int32(0))

                return lax.cond((mk >> g) & 1 != 0, live, lambda m: m, mk)

            mask = lax.fori_loop(0, _NG, gb, mask)
            return t + _TB, mask

        _, mask = lax.while_loop(rare_cond, rare_body, (jnp.int32(_TP), mask))

        def fill_body(g, mk):
            def live(mk):
                _, _, b_rel_v, n_idx = group_geom(g)
                cnt = cnt_v[g, :]
                for slot in range(_K):
                    m = cnt <= slot
                    slot_v = jnp.broadcast_to(slot, (16,)).astype(jnp.int32)
                    plsc.store_scatter(out_v, [b_rel_v, slot_v, n_idx],
                                       inf_v, mask=m)
                return mk

            return lax.cond((mk >> g) & 1 != 0, live, lambda m: m, mk)

        lax.fori_loop(0, _NG, fill_body, mask)

        pltpu.sync_copy(
            out_v, out_hbm.at[pl.ds(b0 + bh, 4), :, pl.ds(n0, 128)])

    return decode


_decoder = _make_decoder()


def kernel(spike_input):
    out = _decoder(spike_input)
    return jnp.transpose(out[:, :_K, :], (1, 0, 2)) * _DT


# speedup vs baseline: 1.0719x; 1.0719x over previous
import functools

import jax
import jax.numpy as jnp
from jax import lax
from jax.experimental import pallas as pl
from jax.experimental.pallas import tpu as pltpu
from jax.experimental.pallas import tpu_sc as plsc

_T = 2048
_B = 64
_N = 256
_K = 16
_DT = 0.001

_TBLK = 32
_NBLK = 3
_SB = 16
_TP = _NBLK * _TBLK
_TB = 32
_NG = 32
_KP = 32


def _make_decoder():
    mesh = plsc.VectorSubcoreMesh(core_axis_name="c", subcore_axis_name="s")

    @functools.partial(
        pl.kernel,
        out_type=jax.ShapeDtypeStruct((_B, _KP, _N), jnp.float32),
        mesh=mesh,
        scratch_types=[
            pltpu.VMEM((_TP, 8, 128), jnp.float32),
            pltpu.VMEM((4, _KP, 128), jnp.float32),
            pltpu.VMEM((_NG, 16), jnp.int32),
            pltpu.SemaphoreType.DMA,
        ],
        compiler_params=pltpu.CompilerParams(needs_layout_passes=False),
    )
    def decode(x_hbm, out_hbm, chunk_v, out_v, cnt_v, sem0):
        core = lax.axis_index("c")
        sub = lax.axis_index("s")
        b0 = pl.multiple_of((sub % 8) * 8, 8)
        n0 = pl.multiple_of((sub // 8) * 128, 128)
        bh = core * 4

        pltpu.async_copy(
            x_hbm.at[pl.ds(0, _TP), pl.ds(b0, 8), pl.ds(n0, 128)],
            chunk_v, sem0).wait()

        lane = jnp.arange(16, dtype=jnp.int32)
        inf_v = jnp.full((16,), jnp.inf, dtype=jnp.float32)
        one_v = jnp.ones((16,), dtype=jnp.int32)
        zero_v = jnp.zeros((16,), dtype=jnp.int32)
        k_v = jnp.full((16,), _K, dtype=jnp.int32)
        onef_v = jnp.ones((16,), dtype=jnp.float32)

        def raw_step(row, geom, cnt, tv):
            b_loc, n_off, b_rel_v, n_idx = geom
            v = chunk_v[row, b_loc, pl.ds(n_off, 16)]
            spike = v > 0.0
            plsc.store_scatter(out_v, [b_rel_v, cnt, n_idx], tv, mask=spike)
            return cnt + jnp.where(spike, one_v, zero_v)

        def num_live(cnt):
            return plsc.all_reduce_population_count(cnt < _K)[0]

        def group_geom(g):
            b_rel = g // 8
            n_off = (g % 8) * 16
            b_loc = bh + b_rel
            b_rel_v = jnp.broadcast_to(b_rel, (16,)).astype(jnp.int32)
            n_idx = n_off + lane
            return b_loc, n_off, b_rel_v, n_idx

        def scan_sub(t0, geom0, geom1, c0, c1, tv):
            @plsc.parallel_loop(t0, t0 + _SB, unroll=_SB, carry=(c0, c1, tv))
            def scan(row, state):
                c0, c1, tv = state
                c0 = raw_step(row, geom0, c0, tv)
                c1 = raw_step(row, geom1, c1, tv)
                return c0, c1, tv + onef_v

            c0, c1, tv = scan
            return jnp.minimum(c0, k_v), jnp.minimum(c1, k_v), tv

        def pair_body(p, mk):
            g0 = p * 2
            g1 = g0 + 1
            geom0 = group_geom(g0)
            geom1 = group_geom(g1)

            def sub_cond(state):
                t, c0, c1, tv = state
                return (t < _TP) & (num_live(c0) + num_live(c1) > 0)

            def sub_body(state):
                t, c0, c1, tv = state
                c0, c1, tv = scan_sub(t, geom0, geom1, c0, c1, tv)
                return t + _SB, c0, c1, tv

            _, c0, c1, _ = lax.while_loop(
                sub_cond, sub_body,
                (jnp.int32(0), zero_v, zero_v,
                 jnp.zeros((16,), jnp.float32)))
            cnt_v[g0, :] = c0
            cnt_v[g1, :] = c1
            s0 = (num_live(c0) > 0).astype(jnp.int32)
            s1 = (num_live(c1) > 0).astype(jnp.int32)
            return mk | (s0 << g0) | (s1 << g1)

        mask = lax.fori_loop(0, _NG // 2, pair_body, jnp.int32(0))

        def rare_cond(state):
            t, mask = state
            return (t < _T) & (mask != 0)

        def rare_body(state):
            t, mask = state
            pltpu.sync_copy(
                x_hbm.at[pl.ds(t, _TB), pl.ds(b0, 8), pl.ds(n0, 128)],
                chunk_v.at[pl.ds(0, _TB)])

            def gb(g, mk):
                def live(mk):
                    geom = group_geom(g)
                    tv0 = jnp.broadcast_to(t.astype(jnp.float32), (16,))

                    def inner(i, state):
                        cnt, tv = state
                        v = chunk_v[i, geom[0], pl.ds(geom[1], 16)]
                        spike = v > 0.0
                        plsc.store_scatter(out_v, [geom[2], cnt, geom[3]],
                                           tv, mask=spike)
                        cnt = jnp.minimum(
                            cnt + jnp.where(spike, one_v, zero_v), k_v)
                        return cnt, tv + onef_v

                    cnt, _ = lax.fori_loop(0, _TB, inner, (cnt_v[g, :], tv0))
                    cnt_v[g, :] = cnt
                    done = num_live(cnt) == 0
                    return mk & ~jnp.where(done, jnp.int32(1) << g,
                                           jnp.int32(0))

                return lax.cond((mk >> g) & 1 != 0, live, lambda m: m, mk)

            mask = lax.fori_loop(0, _NG, gb, mask)
            return t + _TB, mask

        _, mask = lax.while_loop(rare_cond, rare_body, (jnp.int32(_TP), mask))

        def fill_body(g, mk):
            def live(mk):
                _, _, b_rel_v, n_idx = group_geom(g)
                cnt = cnt_v[g, :]
                for slot in range(_K):
                    m = cnt <= slot
                    slot_v = jnp.broadcast_to(slot, (16,)).astype(jnp.int32)
                    plsc.store_scatter(out_v, [b_rel_v, slot_v, n_idx],
                                       inf_v, mask=m)
                return mk

            return lax.cond((mk >> g) & 1 != 0, live, lambda m: m, mk)

        lax.fori_loop(0, _NG, fill_body, mask)

        pltpu.sync_copy(
            out_v, out_hbm.at[pl.ds(b0 + bh, 4), :, pl.ds(n0, 128)])

    return decode


_decoder = _make_decoder()


def kernel(spike_input):
    out = _decoder(spike_input)
    return jnp.transpose(out[:, :_K, :], (1, 0, 2)) * _DT


# speedup vs baseline: 1.0865x; 1.0137x over previous
import functools

import jax
import jax.numpy as jnp
from jax import lax
from jax.experimental import pallas as pl
from jax.experimental.pallas import tpu as pltpu
from jax.experimental.pallas import tpu_sc as plsc

_T = 2048
_B = 64
_N = 256
_K = 16
_DT = 0.001

_TP = 80
_SB = 16
_TB = 16
_NG = 32
_KP = 32


def _make_decoder():
    mesh = plsc.VectorSubcoreMesh(core_axis_name="c", subcore_axis_name="s")

    @functools.partial(
        pl.kernel,
        out_type=jax.ShapeDtypeStruct((_K, _B, _N), jnp.float32),
        mesh=mesh,
        scratch_types=[
            pltpu.VMEM((_TP, 8, 128), jnp.float32),
            pltpu.VMEM((_KP, 4, 128), jnp.float32),
            pltpu.VMEM((_NG, 16), jnp.int32),
            pltpu.VMEM_SHARED((8, 2, _K, 4, 128), jnp.float32),
            pltpu.VMEM((_K // 2, 4, 128), jnp.float32),
            pltpu.VMEM((_K // 2, 4, 128), jnp.float32),
            pltpu.VMEM((_K // 2, 8, 128), jnp.float32),
            pltpu.SemaphoreType.DMA,
        ],
        compiler_params=pltpu.CompilerParams(needs_layout_passes=False),
    )
    def decode(x_hbm, out_hbm, chunk_v, out_v, cnt_v, shared_v, t0_v, t1_v,
               asm_v, sem0):
        core = lax.axis_index("c")
        sub = lax.axis_index("s")
        slab_loc = sub % 8
        half = sub // 8
        b_slab = pl.multiple_of(slab_loc * 8, 8)
        n0 = pl.multiple_of(core * 128, 128)
        bh = half * 4

        pltpu.async_copy(
            x_hbm.at[pl.ds(0, _TP), pl.ds(b_slab, 8), pl.ds(n0, 128)],
            chunk_v, sem0).wait()

        lane = jnp.arange(16, dtype=jnp.int32)
        inf_v = jnp.full((16,), jnp.inf, dtype=jnp.float32)
        one_v = jnp.ones((16,), dtype=jnp.int32)
        zero_v = jnp.zeros((16,), dtype=jnp.int32)
        k_v = jnp.full((16,), _K, dtype=jnp.int32)
        onef_v = jnp.ones((16,), dtype=jnp.float32)
        dt_v = jnp.full((16,), _DT, dtype=jnp.float32)

        def raw_step(row, geom, cnt, val):
            b_loc, n_off, b_rel_v, n_idx = geom
            v = chunk_v[row, b_loc, pl.ds(n_off, 16)]
            spike = v > 0.0
            plsc.store_scatter(out_v, [cnt, b_rel_v, n_idx], val, mask=spike)
            return cnt + jnp.where(spike, one_v, zero_v)

        def num_live(cnt):
            return plsc.all_reduce_population_count(cnt < _K)[0]

        def group_geom(g):
            b_rel = g // 8
            n_off = (g % 8) * 16
            b_loc = bh + b_rel
            b_rel_v = jnp.broadcast_to(b_rel, (16,)).astype(jnp.int32)
            n_idx = n_off + lane
            return b_loc, n_off, b_rel_v, n_idx

        def scan_sub(t0, geom0, geom1, c0, c1, tv):
            @plsc.parallel_loop(t0, t0 + _SB, unroll=_SB, carry=(c0, c1, tv))
            def scan(row, state):
                c0, c1, tv = state
                val = tv * dt_v
                c0 = raw_step(row, geom0, c0, val)
                c1 = raw_step(row, geom1, c1, val)
                return c0, c1, tv + onef_v

            c0, c1, tv = scan
            return jnp.minimum(c0, k_v), jnp.minimum(c1, k_v), tv

        def pair_body(p, mk):
            g0 = p * 2
            g1 = g0 + 1
            geom0 = group_geom(g0)
            geom1 = group_geom(g1)

            def sub_cond(state):
                t, c0, c1, tv = state
                return (t < _TP) & (num_live(c0) + num_live(c1) > 0)

            def sub_body(state):
                t, c0, c1, tv = state
                c0, c1, tv = scan_sub(t, geom0, geom1, c0, c1, tv)
                return t + _SB, c0, c1, tv

            _, c0, c1, _ = lax.while_loop(
                sub_cond, sub_body,
                (jnp.int32(0), zero_v, zero_v,
                 jnp.zeros((16,), jnp.float32)))
            cnt_v[g0, :] = c0
            cnt_v[g1, :] = c1
            s0 = (num_live(c0) > 0).astype(jnp.int32)
            s1 = (num_live(c1) > 0).astype(jnp.int32)
            return mk | (s0 << g0) | (s1 << g1)

        mask = lax.fori_loop(0, _NG // 2, pair_body, jnp.int32(0))

        def rare_cond(state):
            t, mask = state
            return (t < _T) & (mask != 0)

        def rare_body(state):
            t, mask = state
            pltpu.sync_copy(
                x_hbm.at[pl.ds(t, _TB), pl.ds(b_slab, 8), pl.ds(n0, 128)],
                chunk_v.at[pl.ds(0, _TB)])

            def gb(g, mk):
                def live(mk):
                    geom = group_geom(g)
                    tv0 = jnp.broadcast_to(t.astype(jnp.float32), (16,))

                    def inner(i, state):
                        cnt, tv = state
                        v = chunk_v[i, geom[0], pl.ds(geom[1], 16)]
                        spike = v > 0.0
                        plsc.store_scatter(out_v, [cnt, geom[2], geom[3]],
                                           tv * dt_v, mask=spike)
                        cnt = jnp.minimum(
                            cnt + jnp.where(spike, one_v, zero_v), k_v)
                        return cnt, tv + onef_v

                    cnt, _ = lax.fori_loop(0, _TB, inner, (cnt_v[g, :], tv0))
                    cnt_v[g, :] = cnt
                    done = num_live(cnt) == 0
                    return mk & ~jnp.where(done, jnp.int32(1) << g,
                                           jnp.int32(0))

                return lax.cond((mk >> g) & 1 != 0, live, lambda m: m, mk)

            mask = lax.fori_loop(0, _NG, gb, mask)
            return t + _TB, mask

        _, mask = lax.while_loop(rare_cond, rare_body, (jnp.int32(_TP), mask))

        def fill_body(g, mk):
            def live(mk):
                _, _, b_rel_v, n_idx = group_geom(g)
                cnt = cnt_v[g, :]
                for slot in range(_K):
                    m = cnt <= slot
                    slot_v = jnp.broadcast_to(slot, (16,)).astype(jnp.int32)
                    plsc.store_scatter(out_v, [slot_v, b_rel_v, n_idx],
                                       inf_v, mask=m)
                return mk

            return lax.cond((mk >> g) & 1 != 0, live, lambda m: m, mk)

        lax.fori_loop(0, _NG, fill_body, mask)

        pltpu.sync_copy(out_v.at[pl.ds(0, _K)], shared_v.at[slab_loc, half])
        plsc.subcore_barrier()
        slot0 = half * (_K // 2)
        pltpu.sync_copy(shared_v.at[slab_loc, 0, pl.ds(slot0, _K // 2)], t0_v)
        pltpu.sync_copy(shared_v.at[slab_loc, 1, pl.ds(slot0, _K // 2)], t1_v)

        @plsc.parallel_loop(0, _K // 2, carry=jnp.int32(0))
        def _asm(slot, carry):
            for br in range(4):
                for j in range(8):
                    sl = pl.ds(j * 16, 16)
                    asm_v[slot, br, sl] = t0_v[slot, br, sl]
                    asm_v[slot, 4 + br, sl] = t1_v[slot, br, sl]
            return carry

        pltpu.sync_copy(
            asm_v,
            out_hbm.at[pl.ds(slot0, _K // 2), pl.ds(b_slab, 8),
                       pl.ds(n0, 128)])

    return decode


_decoder = _make_decoder()


def kernel(spike_input):
    return _decoder(spike_input)
